# baseline (speedup 1.0000x reference)
import functools

import jax
import jax.numpy as jnp
from jax import lax
from jax.experimental import pallas as pl
from jax.experimental.pallas import tpu as pltpu
from jax.experimental.pallas import tpu_sc as plsc

_VOCAB = 32000
_D = 768
_E = 8
_T = 2048


def _make_emb_gather(V, D, B):
    info = plsc.get_sparse_core_info()
    NC, NS = info.num_cores, info.num_subcores
    NW = NC * NS
    assert B % NW == 0
    b_per_w = B // NW
    mesh = plsc.VectorSubcoreMesh(core_axis_name="c", subcore_axis_name="s")

    @functools.partial(
        pl.kernel, mesh=mesh,
        out_type=jax.ShapeDtypeStruct((B, D), jnp.float32),
        scratch_types=[
            pltpu.VMEM((b_per_w,), jnp.int32),
            pltpu.VMEM((b_per_w, D), jnp.float32),
            pltpu.SemaphoreType.DMA,
        ],
    )
    def k(table_hbm, idx_hbm, out_hbm, idx_v, rows_v, sem):
        wid = lax.axis_index("s") * NC + lax.axis_index("c")
        base = wid * b_per_w
        pltpu.sync_copy(idx_hbm.at[pl.ds(base, b_per_w)], idx_v)
        pltpu.async_copy(table_hbm.at[idx_v], rows_v, sem).wait()
        pltpu.sync_copy(rows_v, out_hbm.at[pl.ds(base, b_per_w)])

    return k


def _moe_dense_body(h_ref, wg_ref, w1_ref, b1_ref, w2_ref, b2_ref, out_ref):
    e = pl.program_id(1)
    h = h_ref[...]
    logits = jnp.dot(h, wg_ref[...], preferred_element_type=jnp.float32)

    eiota = lax.broadcasted_iota(jnp.int32, logits.shape, 1)
    v0 = jnp.max(logits, axis=-1, keepdims=True)
    i0 = jnp.min(jnp.where(logits == v0, eiota, _E), axis=-1, keepdims=True)
    masked = jnp.where(eiota == i0, -jnp.inf, logits)
    v1 = jnp.max(masked, axis=-1, keepdims=True)
    i1 = jnp.min(jnp.where(masked == v1, eiota, _E), axis=-1, keepdims=True)

    ex1 = jnp.exp(v1 - v0)
    w0 = 1.0 / (1.0 + ex1)
    w1 = ex1 / (1.0 + ex1)
    ce = jnp.where(i0 == e, w0, 0.0) + jnp.where(i1 == e, w1, 0.0)
    ce_b = ce.astype(jnp.bfloat16).astype(jnp.float32)

    hid = jnp.maximum(
        jnp.dot(h, w1_ref[0], preferred_element_type=jnp.float32) + b1_ref[0],
        0.0).astype(jnp.bfloat16)
    oute = (jnp.dot(hid, w2_ref[0], preferred_element_type=jnp.float32)
            + b2_ref[0]).astype(jnp.bfloat16)
    contrib = ce_b * oute.astype(jnp.float32)

    @pl.when(e == 0)
    def _():
        out_ref[...] = contrib

    @pl.when(e > 0)
    def _():
        out_ref[...] += contrib


def _moe_dense(h, Wg, W1, b1, W2, b2, bm=256):
    T, D = h.shape
    grid = (T // bm, _E)
    return pl.pallas_call(
        _moe_dense_body,
        grid=grid,
        in_specs=[
            pl.BlockSpec((bm, D), lambda t, e: (t, 0)),
            pl.BlockSpec((D, _E), lambda t, e: (0, 0)),
            pl.BlockSpec((1, D, D), lambda t, e: (e, 0, 0)),
            pl.BlockSpec((1, 1, D), lambda t, e: (e, 0, 0)),
            pl.BlockSpec((1, D, D), lambda t, e: (e, 0, 0)),
            pl.BlockSpec((1, 1, D), lambda t, e: (e, 0, 0)),
        ],
        out_specs=pl.BlockSpec((bm, D), lambda t, e: (t, 0)),
        out_shape=jax.ShapeDtypeStruct((T, D), jnp.float32),
    )(h, Wg, W1, b1.reshape(_E, 1, D), W2, b2.reshape(_E, 1, D))


def _proj_body(h_ref, w_ref, b_ref, out_ref):
    out_ref[...] = (
        jnp.dot(h_ref[...], w_ref[...], preferred_element_type=jnp.float32)
        + b_ref[...]
    )


def _proj(h, Wout, bout2d, bn=1280):
    T, D = h.shape
    V = Wout.shape[1]
    grid = (V // bn,)
    return pl.pallas_call(
        _proj_body,
        grid=grid,
        in_specs=[
            pl.BlockSpec((T, D), lambda n: (0, 0)),
            pl.BlockSpec((D, bn), lambda n: (0, n)),
            pl.BlockSpec((1, bn), lambda n: (0, n)),
        ],
        out_specs=pl.BlockSpec((T, bn), lambda n: (0, n)),
        out_shape=jax.ShapeDtypeStruct((T, V), jnp.float32),
    )(h, Wout, bout2d)


def kernel(x, emb, Wg1, W1a, b1a, W2a, b2a, Wg2, W1b, b1b, W2b, b2b, Wout, bout):
    B, S = x.shape
    bf = jnp.bfloat16
    idx = x.reshape(-1).astype(jnp.int32)
    h32 = _make_emb_gather(_VOCAB, _D, _T)(emb, idx)
    h = h32.astype(bf)
    h = _moe_dense(h, Wg1.astype(bf), W1a.astype(bf), b1a, W2a.astype(bf), b2a)
    h = h.astype(bf)
    h = _moe_dense(h, Wg2.astype(bf), W1b.astype(bf), b1b, W2b.astype(bf), b2b)
    h = h.astype(bf)
    out = _proj(h, Wout.astype(bf), bout.reshape(1, -1))
    return out.reshape(B, S, _VOCAB)


# speedup vs baseline: 1.0842x; 1.0842x over previous
import functools

import jax
import jax.numpy as jnp
from jax import lax
from jax.experimental import pallas as pl
from jax.experimental.pallas import tpu as pltpu
from jax.experimental.pallas import tpu_sc as plsc

_VOCAB = 32000
_D = 768
_E = 8
_T = 2048


def _make_emb_gather(V, D, B):
    info = plsc.get_sparse_core_info()
    NC, NS = info.num_cores, info.num_subcores
    NW = NC * NS
    assert B % NW == 0
    b_per_w = B // NW
    mesh = plsc.VectorSubcoreMesh(core_axis_name="c", subcore_axis_name="s")

    @functools.partial(
        pl.kernel, mesh=mesh,
        out_type=jax.ShapeDtypeStruct((B, D), jnp.float32),
        scratch_types=[
            pltpu.VMEM((b_per_w,), jnp.int32),
            pltpu.VMEM((b_per_w, D), jnp.float32),
            pltpu.SemaphoreType.DMA,
        ],
    )
    def k(table_hbm, idx_hbm, out_hbm, idx_v, rows_v, sem):
        wid = lax.axis_index("s") * NC + lax.axis_index("c")
        base = wid * b_per_w
        pltpu.sync_copy(idx_hbm.at[pl.ds(base, b_per_w)], idx_v)
        pltpu.async_copy(table_hbm.at[idx_v], rows_v, sem).wait()
        pltpu.sync_copy(rows_v, out_hbm.at[pl.ds(base, b_per_w)])

    return k


def _moe_dense_body(h_ref, wg_ref, w1_ref, b1_ref, w2_ref, b2_ref, out_ref):
    e = pl.program_id(1)
    h = h_ref[...]
    logits = jnp.dot(h, wg_ref[...], preferred_element_type=jnp.float32)

    eiota = lax.broadcasted_iota(jnp.int32, logits.shape, 1)
    v0 = jnp.max(logits, axis=-1, keepdims=True)
    i0 = jnp.min(jnp.where(logits == v0, eiota, _E), axis=-1, keepdims=True)
    masked = jnp.where(eiota == i0, -jnp.inf, logits)
    v1 = jnp.max(masked, axis=-1, keepdims=True)
    i1 = jnp.min(jnp.where(masked == v1, eiota, _E), axis=-1, keepdims=True)

    ex1 = jnp.exp(v1 - v0)
    w0 = 1.0 / (1.0 + ex1)
    w1 = ex1 / (1.0 + ex1)
    ce = jnp.where(i0 == e, w0, 0.0) + jnp.where(i1 == e, w1, 0.0)
    ce_b = ce.astype(jnp.bfloat16).astype(jnp.float32)

    hid = jnp.maximum(
        jnp.dot(h, w1_ref[0], preferred_element_type=jnp.float32) + b1_ref[0],
        0.0).astype(jnp.bfloat16)
    oute = (jnp.dot(hid, w2_ref[0], preferred_element_type=jnp.float32)
            + b2_ref[0]).astype(jnp.bfloat16)
    contrib = ce_b * oute.astype(jnp.float32)

    @pl.when(e == 0)
    def _():
        out_ref[...] = contrib

    @pl.when(e > 0)
    def _():
        out_ref[...] += contrib


def _moe_dense(h, Wg, W1, b1, W2, b2, bm=512):
    T, D = h.shape
    grid = (T // bm, _E)
    return pl.pallas_call(
        _moe_dense_body,
        grid=grid,
        in_specs=[
            pl.BlockSpec((bm, D), lambda t, e: (t, 0)),
            pl.BlockSpec((D, _E), lambda t, e: (0, 0)),
            pl.BlockSpec((1, D, D), lambda t, e: (e, 0, 0)),
            pl.BlockSpec((1, 1, D), lambda t, e: (e, 0, 0)),
            pl.BlockSpec((1, D, D), lambda t, e: (e, 0, 0)),
            pl.BlockSpec((1, 1, D), lambda t, e: (e, 0, 0)),
        ],
        out_specs=pl.BlockSpec((bm, D), lambda t, e: (t, 0)),
        out_shape=jax.ShapeDtypeStruct((T, D), jnp.float32),
    )(h, Wg, W1, b1.reshape(_E, 1, D), W2, b2.reshape(_E, 1, D))


def _proj_body(h_ref, w_ref, b_ref, out_ref):
    out_ref[...] = (
        jnp.dot(h_ref[...], w_ref[...], preferred_element_type=jnp.float32)
        + b_ref[...]
    )


def _proj(h, Wout, bout2d, bn=640):
    T, D = h.shape
    V = Wout.shape[1]
    grid = (V // bn,)
    return pl.pallas_call(
        _proj_body,
        grid=grid,
        in_specs=[
            pl.BlockSpec((T, D), lambda n: (0, 0)),
            pl.BlockSpec((D, bn), lambda n: (0, n)),
            pl.BlockSpec((1, bn), lambda n: (0, n)),
        ],
        out_specs=pl.BlockSpec((T, bn), lambda n: (0, n)),
        out_shape=jax.ShapeDtypeStruct((T, V), jnp.float32),
    )(h, Wout, bout2d)


def kernel(x, emb, Wg1, W1a, b1a, W2a, b2a, Wg2, W1b, b1b, W2b, b2b, Wout, bout):
    B, S = x.shape
    bf = jnp.bfloat16
    idx = x.reshape(-1).astype(jnp.int32)
    h32 = _make_emb_gather(_VOCAB, _D, _T)(emb, idx)
    h = h32.astype(bf)
    h = _moe_dense(h, Wg1.astype(bf), W1a.astype(bf), b1a, W2a.astype(bf), b2a)
    h = h.astype(bf)
    h = _moe_dense(h, Wg2.astype(bf), W1b.astype(bf), b1b, W2b.astype(bf), b2b)
    h = h.astype(bf)
    out = _proj(h, Wout.astype(bf), bout.reshape(1, -1))
    return out.reshape(B, S, _VOCAB)
